# baseline (speedup 1.0000x reference)
import jax
import jax.numpy as jnp
from jax.experimental import pallas as pl
from jax.experimental.pallas import tpu as pltpu

_TAU = 0.02
_C2 = 1.4426950408889634 / _TAU

_BLOCK_N = 2000


def _s2r_kernel(qt_ref, v_ref, v2_ref, upd_ref, g_ref,
                m_ref, m2_ref, acc_ref):
    i = pl.program_id(0)
    nsteps = pl.num_programs(0)

    @pl.when(i == 0)
    def _init():
        m_ref[...] = jnp.full_like(m_ref, -1e30)
        m2_ref[...] = jnp.full_like(m2_ref, -1e30)
        acc_ref[...] = jnp.zeros_like(acc_ref)

    m1 = m_ref[...]

    v = v_ref[...]
    s = jax.lax.dot_general(v, qt_ref[...], (((1,), (0,)), ((), ())),
                            preferred_element_type=jnp.float32)
    bm = jnp.max(s, axis=0, keepdims=True)
    m_ref[...] = jnp.maximum(m1, bm)

    v2 = v2_ref[...]
    s2 = jax.lax.dot_general(v2, qt_ref[...], (((1,), (0,)), ((), ())),
                             preferred_element_type=jnp.float32)
    p = jnp.exp2(jnp.minimum((s2 - m1) * _C2, 0.0)
                 ).astype(jnp.bfloat16)
    alpha = jnp.exp2((m2_ref[...] - m1) * _C2)
    va = jnp.concatenate(
        [v2, jnp.ones((v2.shape[0], 1), dtype=v2.dtype)], axis=1
    ).astype(jnp.bfloat16)
    acc_ref[...] = acc_ref[...] * alpha + jax.lax.dot_general(
        va, p, (((0,), (0,)), ((), ())),
        preferred_element_type=jnp.float32)
    m2_ref[...] = m1

    @pl.when(i == nsteps - 1)
    def _fin():
        acc = acc_ref[...]
        msg = acc[:-1, :] / acc[-1:, :]
        upd = qt_ref[...] + msg
        n = jnp.sqrt(jnp.sum(upd * upd, axis=0, keepdims=True))
        upd = upd / jnp.maximum(n, 1e-12)
        upd_ref[...] = upd
        mrow = m_ref[...]
        gmax = jnp.max(mrow, axis=1, keepdims=True)
        ew = jnp.exp2((mrow - gmax) * _C2)
        ew = ew / jnp.sum(ew, axis=1, keepdims=True)
        g = jnp.sum(upd * ew, axis=1, keepdims=True)
        gn = jnp.sqrt(jnp.sum(g * g, axis=(0, 1), keepdims=True))
        g_ref[...] = g / jnp.maximum(gn, 1e-12)


def _build_call(P, D, N, interpret=False):
    nb = N // _BLOCK_N
    return pl.pallas_call(
        _s2r_kernel,
        grid=(nb + 1,),
        in_specs=[
            pl.BlockSpec((D, P), lambda i: (0, 0)),
            pl.BlockSpec((_BLOCK_N, D), lambda i: (jnp.minimum(i, nb - 1), 0)),
            pl.BlockSpec((_BLOCK_N, D), lambda i: (jnp.maximum(i - 1, 0), 0)),
        ],
        out_specs=[
            pl.BlockSpec((D, P), lambda i: (0, 0)),
            pl.BlockSpec((D, 1), lambda i: (0, 0)),
        ],
        out_shape=[
            jax.ShapeDtypeStruct((D, P), jnp.float32),
            jax.ShapeDtypeStruct((D, 1), jnp.float32),
        ],
        scratch_shapes=[
            pltpu.VMEM((1, P), jnp.float32),
            pltpu.VMEM((1, P), jnp.float32),
            pltpu.VMEM((D + 1, P), jnp.float32),
        ],
        interpret=interpret,
    )


@jax.jit
def kernel(test_patches, memory_nodes_gpu):
    P, D = test_patches.shape
    N = memory_nodes_gpu.shape[0]
    qt = test_patches.T
    upd_t, g_t = _build_call(P, D, N)(qt, memory_nodes_gpu,
                                      memory_nodes_gpu)
    return (g_t.T, upd_t.T)


# speedup vs baseline: 1.2826x; 1.2826x over previous
import jax
import jax.numpy as jnp
from jax.experimental import pallas as pl
from jax.experimental.pallas import tpu as pltpu

_TAU = 0.02
_C2 = 1.4426950408889634 / _TAU

_BLOCK_N = 2000


def _s2r_kernel(qt_ref, v_ref, upd_ref, g_ref,
                m_ref, m2_ref, acc_ref, s0_ref, s1_ref, va0_ref, va1_ref):
    i = pl.program_id(0)
    nsteps = pl.num_programs(0)

    @pl.when(i == 0)
    def _init():
        m_ref[...] = jnp.full_like(m_ref, -1e30)
        m2_ref[...] = jnp.full_like(m2_ref, -1e30)
        acc_ref[...] = jnp.zeros_like(acc_ref)
        s1_ref[...] = jnp.full_like(s1_ref, -1e30)
        va1_ref[...] = jnp.zeros_like(va1_ref)

    def _step(sp_ref, vap_ref, sc_ref, vac_ref):
        m1 = m_ref[...]

        v = v_ref[...]
        s = jax.lax.dot_general(v, qt_ref[...], (((1,), (0,)), ((), ())),
                                preferred_element_type=jnp.float32)
        sp_ref[...] = s
        vap_ref[...] = jnp.concatenate(
            [v, jnp.ones((v.shape[0], 1), dtype=v.dtype)], axis=1
        ).astype(jnp.bfloat16)
        bm = jnp.max(s, axis=0, keepdims=True)
        m_ref[...] = jnp.maximum(m1, bm)

        m2 = m2_ref[...]
        p = jnp.exp2((sc_ref[...] - m1) * _C2).astype(jnp.bfloat16)
        alpha = jnp.exp2((m2 - m1) * _C2)
        acc_ref[...] = acc_ref[...] * alpha + jax.lax.dot_general(
            vac_ref[...], p, (((0,), (0,)), ((), ())),
            preferred_element_type=jnp.float32)
        m2_ref[...] = m1

    @pl.when(i % 2 == 0)
    def _even():
        _step(s0_ref, va0_ref, s1_ref, va1_ref)

    @pl.when(i % 2 == 1)
    def _odd():
        _step(s1_ref, va1_ref, s0_ref, va0_ref)

    @pl.when(i == nsteps - 1)
    def _fin():
        acc = acc_ref[...]
        msg = acc[:-1, :] / acc[-1:, :]
        upd = qt_ref[...] + msg
        n = jnp.sqrt(jnp.sum(upd * upd, axis=0, keepdims=True))
        upd = upd / jnp.maximum(n, 1e-12)
        upd_ref[...] = upd
        mrow = m_ref[...]
        gmax = jnp.max(mrow, axis=1, keepdims=True)
        ew = jnp.exp2((mrow - gmax) * _C2)
        ew = ew / jnp.sum(ew, axis=1, keepdims=True)
        g = jnp.sum(upd * ew, axis=1, keepdims=True)
        gn = jnp.sqrt(jnp.sum(g * g, axis=(0, 1), keepdims=True))
        g_ref[...] = g / jnp.maximum(gn, 1e-12)


def _build_call(P, D, N, interpret=False):
    nb = N // _BLOCK_N
    return pl.pallas_call(
        _s2r_kernel,
        grid=(nb + 1,),
        in_specs=[
            pl.BlockSpec((D, P), lambda i: (0, 0)),
            pl.BlockSpec((_BLOCK_N, D), lambda i: (jnp.minimum(i, nb - 1), 0)),
        ],
        out_specs=[
            pl.BlockSpec((D, P), lambda i: (0, 0)),
            pl.BlockSpec((D, 1), lambda i: (0, 0)),
        ],
        out_shape=[
            jax.ShapeDtypeStruct((D, P), jnp.float32),
            jax.ShapeDtypeStruct((D, 1), jnp.float32),
        ],
        scratch_shapes=[
            pltpu.VMEM((1, P), jnp.float32),
            pltpu.VMEM((1, P), jnp.float32),
            pltpu.VMEM((D + 1, P), jnp.float32),
            pltpu.VMEM((_BLOCK_N, P), jnp.float32),
            pltpu.VMEM((_BLOCK_N, P), jnp.float32),
            pltpu.VMEM((_BLOCK_N, D + 1), jnp.bfloat16),
            pltpu.VMEM((_BLOCK_N, D + 1), jnp.bfloat16),
        ],
        interpret=interpret,
    )


@jax.jit
def kernel(test_patches, memory_nodes_gpu):
    P, D = test_patches.shape
    N = memory_nodes_gpu.shape[0]
    qt = test_patches.T
    upd_t, g_t = _build_call(P, D, N)(qt, memory_nodes_gpu)
    return (g_t.T, upd_t.T)
